# baseline (speedup 1.0000x reference)
import functools

import jax
import jax.numpy as jnp
from jax import lax
from jax.experimental import pallas as pl
from jax.experimental.pallas import tpu as pltpu
from jax.experimental.pallas import tpu_sc as plsc

_NUM_WORKERS = 32
_CH = 128


def kernel(input, table):
    flat = input.reshape(-1).astype(jnp.int32)
    b_total = flat.shape[0]
    d = table.shape[1]
    bpw = b_total // _NUM_WORKERS
    n_chunks = bpw // _CH
    n_bands = d // 8
    tcols = b_total // _CH

    mesh = plsc.VectorSubcoreMesh(core_axis_name="c", subcore_axis_name="s")

    @functools.partial(
        pl.kernel,
        mesh=mesh,
        compiler_params=pltpu.CompilerParams(use_tc_tiling_on_sc=False, needs_layout_passes=False),
        out_type=jax.ShapeDtypeStruct((n_bands, tcols, 8, _CH), jnp.float32),
        scratch_types=[
            pltpu.VMEM((bpw,), jnp.int32),
            pltpu.VMEM((_CH, d), jnp.float32),
            pltpu.VMEM((_CH, d), jnp.float32),
            pltpu.VMEM((d, _CH), jnp.float32),
            pltpu.VMEM((d, _CH), jnp.float32),
            pltpu.SemaphoreType.DMA,
            pltpu.SemaphoreType.DMA,
            pltpu.SemaphoreType.DMA,
            pltpu.SemaphoreType.DMA,
        ],
    )
    def gather_kernel(idx_hbm, table_hbm, outp_hbm, idx_v,
                      rows0, rows1, tb0, tb1, g0, g1, w0, w1):
        rows = (rows0, rows1)
        tbuf = (tb0, tb1)
        gsem = (g0, g1)
        wsem = (w0, w1)

        wid = lax.axis_index("s") * 2 + lax.axis_index("c")
        base = wid * bpw
        tcol0 = wid * n_chunks
        pltpu.sync_copy(idx_hbm.at[pl.ds(base, bpw)], idx_v)

        iota = lax.iota(jnp.int32, 16)
        row_ids = [iota + 16 * k for k in range(_CH // 16)]

        def gather(t, b):
            return pltpu.async_copy(
                table_hbm.at[idx_v.at[pl.ds(t * _CH, _CH)]], rows[b], gsem[b])

        def out_writes(t, b):
            return [
                pltpu.make_async_copy(
                    tbuf[b].at[pl.ds(a * 8, 8)],
                    outp_hbm.at[a, tcol0 + t],
                    wsem[b])
                for a in range(n_bands)
            ]

        def transpose_chunk(b):
            def dim_body(dim, carry):
                col = jnp.zeros((16,), jnp.int32) + dim
                for k in range(_CH // 16):
                    v = plsc.load_gather(rows[b], [row_ids[k], col])
                    tbuf[b][dim, pl.ds(k * 16, 16)] = v
                return carry
            lax.fori_loop(0, d, dim_body, 0)

        g_pending = [gather(0, 0), gather(1, 1)]
        for b in range(2):
            g_pending[b].wait()
            transpose_chunk(b)
            for c in out_writes(b, b):
                c.start()
            if 2 + b < n_chunks:
                gather(2 + b, b)

        def outer(s, carry):
            for b in range(2):
                t = s * 2 + b
                pltpu.make_async_copy(
                    table_hbm.at[idx_v.at[pl.ds(t * _CH, _CH)]],
                    rows[b], gsem[b]).wait()
                for c in out_writes(t - 2, b):
                    c.wait()
                transpose_chunk(b)
                for c in out_writes(t, b):
                    c.start()

                @pl.when(t + 2 < n_chunks)
                def _():
                    gather(t + 2, b)
            return carry

        lax.fori_loop(1, n_chunks // 2, outer, 0)

        for b in range(2):
            for c in out_writes(n_chunks - 2 + b, b):
                c.wait()

    outp = gather_kernel(flat, table)
    return outp.transpose(1, 3, 0, 2).reshape(b_total, d)


# speedup vs baseline: 1.5290x; 1.5290x over previous
import functools

import jax
import jax.numpy as jnp
from jax import lax
from jax.experimental import pallas as pl
from jax.experimental.pallas import tpu as pltpu
from jax.experimental.pallas import tpu_sc as plsc

_NUM_WORKERS = 32
_CH = 128


def kernel(input, table):
    flat = input.reshape(-1).astype(jnp.int32)
    b_total = flat.shape[0]
    d = table.shape[1]
    bpw = b_total // _NUM_WORKERS
    n_chunks = bpw // _CH
    n_bands = d // 8
    tcols = b_total // _CH

    mesh = plsc.VectorSubcoreMesh(core_axis_name="c", subcore_axis_name="s")

    @functools.partial(
        pl.kernel,
        mesh=mesh,
        compiler_params=pltpu.CompilerParams(use_tc_tiling_on_sc=False, needs_layout_passes=False),
        out_type=jax.ShapeDtypeStruct((n_bands, tcols, 8, _CH), jnp.float32),
        scratch_types=[
            pltpu.VMEM((bpw,), jnp.int32),
            pltpu.VMEM((_CH, d), jnp.float32),
            pltpu.VMEM((_CH, d), jnp.float32),
            pltpu.VMEM((d, _CH), jnp.float32),
            pltpu.VMEM((d, _CH), jnp.float32),
            pltpu.SemaphoreType.DMA,
            pltpu.SemaphoreType.DMA,
            pltpu.SemaphoreType.DMA,
            pltpu.SemaphoreType.DMA,
        ],
    )
    def gather_kernel(idx_hbm, table_hbm, outp_hbm, idx_v,
                      rows0, rows1, tb0, tb1, g0, g1, w0, w1):
        rows = (rows0, rows1)
        tbuf = (tb0, tb1)
        gsem = (g0, g1)
        wsem = (w0, w1)

        wid = lax.axis_index("s") * 2 + lax.axis_index("c")
        base = wid * bpw
        tcol0 = wid * n_chunks
        pltpu.sync_copy(idx_hbm.at[pl.ds(base, bpw)], idx_v)

        iota = lax.iota(jnp.int32, 16)
        row_ids = [iota + 16 * k for k in range(_CH // 16)]

        def gather(t, b):
            return pltpu.async_copy(
                table_hbm.at[idx_v.at[pl.ds(t * _CH, _CH)]], rows[b], gsem[b])

        def out_writes(t, b):
            return [
                pltpu.make_async_copy(
                    tbuf[b].at[pl.ds(a * 8, 8)],
                    outp_hbm.at[a, tcol0 + t],
                    wsem[b])
                for a in range(n_bands)
            ]

        def transpose_chunk(b):
            @plsc.parallel_loop(0, d, 1, unroll=8)
            def _(dim):
                col = jnp.zeros((16,), jnp.int32) + dim
                for k in range(_CH // 16):
                    v = plsc.load_gather(rows[b], [row_ids[k], col])
                    tbuf[b][dim, pl.ds(k * 16, 16)] = v

        g_pending = [gather(0, 0), gather(1, 1)]
        for b in range(2):
            g_pending[b].wait()
            transpose_chunk(b)
            for c in out_writes(b, b):
                c.start()
            if 2 + b < n_chunks:
                gather(2 + b, b)

        def outer(s, carry):
            for b in range(2):
                t = s * 2 + b
                pltpu.make_async_copy(
                    table_hbm.at[idx_v.at[pl.ds(t * _CH, _CH)]],
                    rows[b], gsem[b]).wait()
                for c in out_writes(t - 2, b):
                    c.wait()
                transpose_chunk(b)
                for c in out_writes(t, b):
                    c.start()

                @pl.when(t + 2 < n_chunks)
                def _():
                    gather(t + 2, b)
            return carry

        lax.fori_loop(1, n_chunks // 2, outer, 0)

        for b in range(2):
            for c in out_writes(n_chunks - 2 + b, b):
                c.wait()

    outp = gather_kernel(flat, table)
    return outp.transpose(1, 3, 0, 2).reshape(b_total, d)


# speedup vs baseline: 1.5749x; 1.0300x over previous
import functools

import jax
import jax.numpy as jnp
from jax import lax
from jax.experimental import pallas as pl
from jax.experimental.pallas import tpu as pltpu
from jax.experimental.pallas import tpu_sc as plsc

_NUM_WORKERS = 32
_CH = 128


def kernel(input, table):
    flat = input.reshape(-1).astype(jnp.int32)
    b_total = flat.shape[0]
    d = table.shape[1]
    bpw = b_total // _NUM_WORKERS
    n_chunks = bpw // _CH
    n_bands = d // 8
    tcols = b_total // _CH

    mesh = plsc.VectorSubcoreMesh(core_axis_name="c", subcore_axis_name="s")

    @functools.partial(
        pl.kernel,
        mesh=mesh,
        compiler_params=pltpu.CompilerParams(use_tc_tiling_on_sc=False, needs_layout_passes=False),
        out_type=jax.ShapeDtypeStruct((n_bands, tcols, 8, _CH), jnp.float32),
        scratch_types=[
            pltpu.VMEM((bpw,), jnp.int32),
            pltpu.VMEM((_CH, d), jnp.float32),
            pltpu.VMEM((_CH, d), jnp.float32),
            pltpu.VMEM((d, _CH), jnp.float32),
            pltpu.VMEM((d, _CH), jnp.float32),
            pltpu.SemaphoreType.DMA,
            pltpu.SemaphoreType.DMA,
            pltpu.SemaphoreType.DMA,
            pltpu.SemaphoreType.DMA,
        ],
    )
    def gather_kernel(idx_hbm, table_hbm, outp_hbm, idx_v,
                      rows0, rows1, tb0, tb1, g0, g1, w0, w1):
        rows = (rows0, rows1)
        tbuf = (tb0, tb1)
        gsem = (g0, g1)
        wsem = (w0, w1)

        wid = lax.axis_index("s") * 2 + lax.axis_index("c")
        base = wid * bpw
        tcol0 = wid * n_chunks
        pltpu.sync_copy(idx_hbm.at[pl.ds(base, bpw)], idx_v)

        iota = lax.iota(jnp.int32, 16)
        row_ids = [iota + 16 * k for k in range(_CH // 16)]

        def gather(t, b):
            return pltpu.async_copy(
                table_hbm.at[idx_v.at[pl.ds(t * _CH, _CH)]], rows[b], gsem[b])

        def out_writes(t, b):
            return [
                pltpu.make_async_copy(
                    tbuf[b].at[pl.ds(a * 8, 8)],
                    outp_hbm.at[a, tcol0 + t],
                    wsem[b])
                for a in range(n_bands)
            ]

        def transpose_chunk(b):
            @plsc.parallel_loop(0, d, 1, unroll=8)
            def _(dim):
                col = jnp.zeros((16,), jnp.int32) + dim
                for k in range(_CH // 16):
                    v = plsc.load_gather(rows[b], [row_ids[k], col])
                    tbuf[b][dim, pl.ds(k * 16, 16)] = v

        g_pending = [gather(0, 0), gather(1, 1)]
        for b in range(2):
            g_pending[b].wait()
            transpose_chunk(b)
            for c in out_writes(b, b):
                c.start()
            if 2 + b < n_chunks:
                gather(2 + b, b)

        def outer(s, carry):
            for b in range(2):
                t = s * 2 + b
                pltpu.make_async_copy(
                    table_hbm.at[idx_v.at[pl.ds(t * _CH, _CH)]],
                    rows[b], gsem[b]).wait()
                for c in out_writes(t - 2, b):
                    c.wait()
                transpose_chunk(b)
                for c in out_writes(t, b):
                    c.start()

                @pl.when(t + 2 < n_chunks)
                def _():
                    gather(t + 2, b)
            return carry

        lax.fori_loop(1, n_chunks // 2, outer, 0)

        for b in range(2):
            for c in out_writes(n_chunks - 2 + b, b):
                c.wait()

    tbl2 = jnp.pad(table, ((0, 0), (0, d))).reshape(2 * table.shape[0], d)
    outp = gather_kernel(flat * 2, tbl2)
    return outp.transpose(1, 3, 0, 2).reshape(b_total, d)
